# baseline (speedup 1.0000x reference)
import functools

import jax
import jax.numpy as jnp
from jax import lax
from jax.experimental import pallas as pl
from jax.experimental.pallas import tpu as pltpu
from jax.experimental.pallas import tpu_sc as plsc

N = 10000
F = 128
E = 320000
NC = 2
NS = 16
L = 16
BLK = 128
NQ = 8
EPT = E // NS // NQ
NBLK = -(-EPT // BLK)
EPTP = NBLK * BLK
NBLK1 = NQ * NBLK
EPTP1 = NQ * EPTP
RB = 2000

_P = lax.Precision.HIGHEST
_f32 = jnp.float32


RBP = 2000


def _prep_body(xd, xt, wdd, wtt, wx, qa, qc, qx,
               tab1, tab2, eea, eec, eeb, eed, mx):
    i = pl.program_id(0)
    a = jnp.dot(xd[...], wdd[...], preferred_element_type=_f32, precision=_P)
    c = jnp.dot(xt[...], wtt[...], preferred_element_type=_f32, precision=_P)
    b = jnp.dot(xt[...], wx[...], preferred_element_type=_f32, precision=_P)
    d = jnp.dot(xd[...], wx[...], preferred_element_type=_f32, precision=_P)
    tab1[0] = a
    tab1[1] = c
    tab2[0] = b
    tab2[1] = d
    va = jnp.dot(a, qa[...], preferred_element_type=_f32, precision=_P)
    vc = jnp.dot(c, qc[...], preferred_element_type=_f32, precision=_P)
    vb = jnp.dot(b, qx[...], preferred_element_type=_f32, precision=_P)
    vd = jnp.dot(d, qx[...], preferred_element_type=_f32, precision=_P)
    eea[...] = va
    eec[...] = vc
    eeb[...] = vb
    eed[...] = vd

    @pl.when(i == 0)
    def _init():
        mx[...] = jnp.full((8, 128), -jnp.inf, _f32)

    for r, v in enumerate((va, vc, vb, vd)):
        for col in range(2):
            m = jnp.max(v[:, col:col + 1])
            row = 2 * r + col
            mx[row:row + 1, :] = jnp.maximum(mx[row:row + 1, :],
                                             jnp.full((1, 128), m, _f32))


def _prep(xd, xt, wdd, wtt, wx, qa, qc, qx):
    sds = jax.ShapeDtypeStruct
    x_spec = pl.BlockSpec((RBP, F), lambda i: (i, 0))
    w_spec = pl.BlockSpec((F, F), lambda i: (0, 0))
    q_spec = pl.BlockSpec((F, 2), lambda i: (0, 0))
    tab_spec = pl.BlockSpec((2, RBP, F), lambda i: (0, i, 0))
    ee_spec = pl.BlockSpec((RBP, 2), lambda i: (i, 0))
    mx_spec = pl.BlockSpec((8, 128), lambda i: (0, 0))
    return pl.pallas_call(
        _prep_body,
        grid=(N // RBP,),
        in_specs=[x_spec, x_spec, w_spec, w_spec, w_spec,
                  q_spec, q_spec, q_spec],
        out_specs=(tab_spec, tab_spec, ee_spec, ee_spec, ee_spec, ee_spec,
                   mx_spec),
        out_shape=(sds((2, N, F), _f32), sds((2, N, F), _f32),
                   sds((N, 2), _f32), sds((N, 2), _f32),
                   sds((N, 2), _f32), sds((N, 2), _f32),
                   sds((8, 128), _f32)),
    )(xd, xt, wdd, wtt, wx, qa, qc, qx)


def _w_body(el_hbm, er_hbm, m_hbm, src_hbm, dst_hbm, zs_hbm,
            ex_out, s_out,
            elv, erv, mv, srcv, dstv, exv, s_loc):
    cid = lax.axis_index("c")
    sid = lax.axis_index("s")
    wid = cid * NS + sid

    pltpu.sync_copy(el_hbm.at[cid], elv)
    pltpu.sync_copy(er_hbm.at[cid], erv)
    pltpu.sync_copy(m_hbm.at[cid], mv)
    pltpu.sync_copy(src_hbm.at[wid], srcv)
    pltpu.sync_copy(dst_hbm.at[wid], dstv)
    pltpu.sync_copy(zs_hbm, s_loc)

    srcoff = cid * jnp.int32(N)

    def blk(j, carry):
        m16 = mv[...]
        for k in range(BLK // L):
            s16 = srcv[j, pl.ds(k * L, L)] - srcoff
            d16 = dstv[j, pl.ds(k * L, L)]
            z = (plsc.load_gather(elv, [s16])
                 + plsc.load_gather(erv, [d16]))
            z = jnp.where(z >= 0, z, z * jnp.float32(0.2))
            pos = lax.rem(j, NBLK) * BLK + k * L + lax.iota(jnp.int32, L)
            ex16 = jnp.where(pos < EPT, jnp.exp(z - m16), jnp.float32(0.0))
            exv[j, pl.ds(k * L, L)] = ex16
            plsc.addupdate_scatter(s_loc, [d16], ex16)
        return carry

    lax.fori_loop(0, NBLK1, blk, 0)

    pltpu.sync_copy(exv, ex_out.at[cid, sid])
    pltpu.sync_copy(s_loc, s_out.at[cid, sid])


def _sc_weights(el2, er2, m2, src, dst, zs):
    sds = jax.ShapeDtypeStruct
    return pl.kernel(
        _w_body,
        out_type=(sds((NC, NS, NBLK1, BLK), _f32), sds((NC, NS, N), _f32)),
        mesh=plsc.VectorSubcoreMesh(core_axis_name="c", subcore_axis_name="s"),
        compiler_params=pltpu.CompilerParams(needs_layout_passes=False),
        scratch_types=[
            pltpu.VMEM((N,), _f32),
            pltpu.VMEM((N,), _f32),
            pltpu.VMEM((L,), _f32),
            pltpu.VMEM((NBLK1, BLK), jnp.int32),
            pltpu.VMEM((NBLK1, BLK), jnp.int32),
            pltpu.VMEM((NBLK1, BLK), _f32),
            pltpu.VMEM((N,), _f32),
        ],
    )(el2, er2, m2, src, dst, zs)


def _a_body(tab_hbm, ex_hbm, src_hbm, dst_hbm, accin_hbm,
            acc_out,
            acc_sh, srcv, dstv, exv, rowsv, sem, ssem):
    cid = lax.axis_index("c")
    sid = lax.axis_index("s")
    wid = cid * NS + sid

    pltpu.sync_copy(src_hbm.at[wid], srcv)
    pltpu.sync_copy(dst_hbm.at[wid], dstv)
    pltpu.sync_copy(ex_hbm.at[wid], exv)

    @pl.when(sid == 0)
    def _restore():
        pltpu.sync_copy(accin_hbm.at[cid], acc_sh)

    plsc.subcore_barrier()

    pltpu.async_copy(tab_hbm.at[srcv.at[0]], rowsv.at[0], sem.at[0])

    def blk(j, carry):
        p = lax.rem(j, 2)
        pltpu.make_async_copy(tab_hbm.at[srcv.at[j]], rowsv.at[p],
                              sem.at[p]).wait()

        for k in range(BLK // L):
            for lq in range(L):
                e = k * L + lq
                bc = plsc.load_gather(
                    exv, [jnp.full((L,), j, jnp.int32),
                          jnp.full((L,), e, jnp.int32)])
                for col in range(F // L):
                    rowsv[p, e, pl.ds(col * L, L)] = (
                        rowsv[p, e, pl.ds(col * L, L)] * bc)
        pltpu.async_copy(rowsv.at[p], acc_sh.at[dstv.at[j]], ssem.at[p],
                         add=True)

        @pl.when(j >= 1)
        def _drain():
            pltpu.make_async_copy(rowsv.at[1 - p],
                                  acc_sh.at[dstv.at[j - 1]],
                                  ssem.at[1 - p]).wait()

        @pl.when(j + 1 < NBLK)
        def _prefetch():
            pltpu.async_copy(tab_hbm.at[srcv.at[j + 1]], rowsv.at[1 - p],
                             sem.at[1 - p])

        return carry

    lax.fori_loop(0, NBLK, blk, 0)
    pltpu.make_async_copy(rowsv.at[(NBLK - 1) % 2],
                          acc_sh.at[dstv.at[NBLK - 1]],
                          ssem.at[(NBLK - 1) % 2]).wait()

    plsc.subcore_barrier()

    @pl.when(sid == 0)
    def _flush():
        pltpu.sync_copy(acc_sh, acc_out.at[cid])


def _sc_accum(tab, exq, src, dst, accin):
    sds = jax.ShapeDtypeStruct
    return pl.kernel(
        _a_body,
        out_type=sds((NC, N, F), _f32),
        mesh=plsc.VectorSubcoreMesh(core_axis_name="c", subcore_axis_name="s"),
        compiler_params=pltpu.CompilerParams(needs_layout_passes=False),
        scratch_types=[
            pltpu.VMEM_SHARED((N, F), _f32),
            pltpu.VMEM((NBLK, BLK), jnp.int32),
            pltpu.VMEM((NBLK, BLK), jnp.int32),
            pltpu.VMEM((NBLK, BLK), _f32),
            pltpu.VMEM((2, BLK, F), _f32),
            pltpu.SemaphoreType.DMA((2,)),
            pltpu.SemaphoreType.DMA((2,)),
        ],
    )(tab, exq, src, dst, accin)


def _combine_body(acc1, s1, acc2, s2, bdd, btt, bx, hd, ht):
    def rel_term(acc_ref, s_ref, rel):
        sv = jnp.sum(s_ref[rel], axis=1).reshape(RB, 1)
        return jnp.where(sv > 0, acc_ref[rel] / sv, jnp.float32(0.0))

    hd[...] = (rel_term(acc1, s1, 0) + rel_term(acc2, s2, 0)
               + bdd[...] + bx[...])
    ht[...] = (rel_term(acc1, s1, 1) + rel_term(acc2, s2, 1)
               + btt[...] + bx[...])


def _combine(acc1, s1, acc2, s2, bdd, btt, bx):
    sds = jax.ShapeDtypeStruct
    acc_spec = pl.BlockSpec((NC, RB, F), lambda i: (0, i, 0))
    s_spec = pl.BlockSpec((NC, RB, NS), lambda i: (0, i, 0))
    b_spec = pl.BlockSpec((F,), lambda i: (0,))
    out_spec = pl.BlockSpec((RB, F), lambda i: (i, 0))
    return pl.pallas_call(
        _combine_body,
        grid=(N // RB,),
        in_specs=[acc_spec, s_spec, acc_spec, s_spec, b_spec, b_spec, b_spec],
        out_specs=(out_spec, out_spec),
        out_shape=(sds((N, F), _f32), sds((N, F), _f32)),
    )(acc1, s1, acc2, s2, bdd, btt, bx)


def _edges4(e_src, offset):
    e2 = jnp.pad(e_src.reshape(NS, NQ, EPT), ((0, 0), (0, 0), (0, EPTP - EPT)))
    return (e2 + jnp.int32(offset)).reshape(NS, NQ, NBLK, BLK)


def kernel(x_d, x_t, edge_dd, edge_tt, edge_dt, edge_td,
           W_dd, al_dd, ar_dd, b_dd,
           W_tt, al_tt, ar_tt, b_tt,
           W_x, al_x, ar_x, b_x):
    qa = jnp.stack([al_dd, ar_dd], axis=1)
    qc = jnp.stack([al_tt, ar_tt], axis=1)
    qx = jnp.stack([al_x, ar_x], axis=1)

    tab1, tab2, eea, eec, eeb, eed, mx = _prep(
        x_d, x_t, W_dd, W_tt, W_x, qa, qc, qx)
    tab1 = tab1.reshape(2 * N, F)
    tab2 = tab2.reshape(2 * N, F)

    el1 = jnp.stack([eea[:, 0], eec[:, 0]])
    er1 = jnp.stack([eea[:, 1], eec[:, 1]])
    m1 = jnp.stack([
        jnp.broadcast_to(mx[0, 0] + mx[1, 0], (L,)),
        jnp.broadcast_to(mx[2, 0] + mx[3, 0], (L,)),
    ])
    src1 = jnp.concatenate([_edges4(edge_dd[0], 0), _edges4(edge_tt[0], N)])
    dst1 = jnp.concatenate([_edges4(edge_dd[1], 0), _edges4(edge_tt[1], 0)])

    el2 = jnp.stack([eeb[:, 0], eed[:, 0]])
    er2 = jnp.stack([eed[:, 1], eeb[:, 1]])
    m2 = jnp.stack([
        jnp.broadcast_to(mx[4, 0] + mx[7, 0], (L,)),
        jnp.broadcast_to(mx[6, 0] + mx[5, 0], (L,)),
    ])
    src2 = jnp.concatenate([_edges4(edge_td[0], 0), _edges4(edge_dt[0], N)])
    dst2 = jnp.concatenate([_edges4(edge_td[1], 0), _edges4(edge_dt[1], 0)])

    zs = jnp.zeros((N,), _f32)
    zacc = jnp.zeros((NC, N, F), _f32)

    outs = []
    for tab, el, er, m, src, dst in ((tab1, el1, er1, m1, src1, dst1),
                                     (tab2, el2, er2, m2, src2, dst2)):
        if outs:
            pa, ps = outs[-1]
            tab, el, er, m, pa, ps = lax.optimization_barrier(
                (tab, el, er, m, pa, ps))
            outs[-1] = (pa, ps)
        exq, s_parts = _sc_weights(
            el, er, m,
            src.reshape(NC * NS, NBLK1, BLK),
            dst.reshape(NC * NS, NBLK1, BLK), zs)
        acc = zacc
        exq4 = exq.reshape(NC * NS, NQ, NBLK, BLK)
        src4 = src.reshape(NC * NS, NQ, NBLK, BLK)
        dst4 = dst.reshape(NC * NS, NQ, NBLK, BLK)
        for q in range(NQ):
            acc = _sc_accum(tab, exq4[:, q], src4[:, q], dst4[:, q], acc)
        outs.append((acc, s_parts))

    (acc1, s1), (acc2, s2) = outs
    s1 = s1.transpose(0, 2, 1)
    s2 = s2.transpose(0, 2, 1)
    return _combine(acc1, s1, acc2, s2, b_dd, b_tt, b_x)


# speedup vs baseline: 1.1167x; 1.1167x over previous
import functools

import jax
import jax.numpy as jnp
from jax import lax
from jax.experimental import pallas as pl
from jax.experimental.pallas import tpu as pltpu
from jax.experimental.pallas import tpu_sc as plsc

N = 10000
F = 128
E = 320000
NC = 2
NS = 16
L = 16
BLK = 128
NQ = 8
EPT = E // NS // NQ
NBLK = -(-EPT // BLK)
EPTP = NBLK * BLK
NBLK1 = NQ * NBLK
EPTP1 = NQ * EPTP
RB = 2000

_P = lax.Precision.HIGHEST
_f32 = jnp.float32


RBP = 2000


def _prep_body(xd, xt, wdd, wtt, wx, qa, qc, qx,
               tab1, tab2, eea, eec, eeb, eed, mx):
    i = pl.program_id(0)
    a = jnp.dot(xd[...], wdd[...], preferred_element_type=_f32, precision=_P)
    c = jnp.dot(xt[...], wtt[...], preferred_element_type=_f32, precision=_P)
    b = jnp.dot(xt[...], wx[...], preferred_element_type=_f32, precision=_P)
    d = jnp.dot(xd[...], wx[...], preferred_element_type=_f32, precision=_P)
    tab1[0] = a
    tab1[1] = c
    tab2[0] = b
    tab2[1] = d
    va = jnp.dot(a, qa[...], preferred_element_type=_f32, precision=_P)
    vc = jnp.dot(c, qc[...], preferred_element_type=_f32, precision=_P)
    vb = jnp.dot(b, qx[...], preferred_element_type=_f32, precision=_P)
    vd = jnp.dot(d, qx[...], preferred_element_type=_f32, precision=_P)
    eea[...] = va
    eec[...] = vc
    eeb[...] = vb
    eed[...] = vd

    @pl.when(i == 0)
    def _init():
        mx[...] = jnp.full((8, 128), -jnp.inf, _f32)

    for r, v in enumerate((va, vc, vb, vd)):
        for col in range(2):
            m = jnp.max(v[:, col:col + 1])
            row = 2 * r + col
            mx[row:row + 1, :] = jnp.maximum(mx[row:row + 1, :],
                                             jnp.full((1, 128), m, _f32))


def _prep(xd, xt, wdd, wtt, wx, qa, qc, qx):
    sds = jax.ShapeDtypeStruct
    x_spec = pl.BlockSpec((RBP, F), lambda i: (i, 0))
    w_spec = pl.BlockSpec((F, F), lambda i: (0, 0))
    q_spec = pl.BlockSpec((F, 2), lambda i: (0, 0))
    tab_spec = pl.BlockSpec((2, RBP, F), lambda i: (0, i, 0))
    ee_spec = pl.BlockSpec((RBP, 2), lambda i: (i, 0))
    mx_spec = pl.BlockSpec((8, 128), lambda i: (0, 0))
    return pl.pallas_call(
        _prep_body,
        grid=(N // RBP,),
        in_specs=[x_spec, x_spec, w_spec, w_spec, w_spec,
                  q_spec, q_spec, q_spec],
        out_specs=(tab_spec, tab_spec, ee_spec, ee_spec, ee_spec, ee_spec,
                   mx_spec),
        out_shape=(sds((2, N, F), _f32), sds((2, N, F), _f32),
                   sds((N, 2), _f32), sds((N, 2), _f32),
                   sds((N, 2), _f32), sds((N, 2), _f32),
                   sds((8, 128), _f32)),
    )(xd, xt, wdd, wtt, wx, qa, qc, qx)


def _w_body(el_hbm, er_hbm, m_hbm, src_hbm, dst_hbm, zs_hbm,
            ex_out, s_out,
            elv, erv, mv, srcv, dstv, exv, s_loc):
    cid = lax.axis_index("c")
    sid = lax.axis_index("s")
    wid = cid * NS + sid

    pltpu.sync_copy(el_hbm.at[cid], elv)
    pltpu.sync_copy(er_hbm.at[cid], erv)
    pltpu.sync_copy(m_hbm.at[cid], mv)
    pltpu.sync_copy(src_hbm.at[wid], srcv)
    pltpu.sync_copy(dst_hbm.at[wid], dstv)
    pltpu.sync_copy(zs_hbm, s_loc)

    srcoff = cid * jnp.int32(N)

    def blk(j, carry):
        m16 = mv[...]
        for k in range(BLK // L):
            s16 = srcv[j, pl.ds(k * L, L)] - srcoff
            d16 = dstv[j, pl.ds(k * L, L)]
            z = (plsc.load_gather(elv, [s16])
                 + plsc.load_gather(erv, [d16]))
            z = jnp.where(z >= 0, z, z * jnp.float32(0.2))
            pos = lax.rem(j, NBLK) * BLK + k * L + lax.iota(jnp.int32, L)
            ex16 = jnp.where(pos < EPT, jnp.exp(z - m16), jnp.float32(0.0))
            exv[j, pl.ds(k * L, L)] = ex16
            plsc.addupdate_scatter(s_loc, [d16], ex16)
        return carry

    lax.fori_loop(0, NBLK1, blk, 0)

    pltpu.sync_copy(exv, ex_out.at[cid, sid])
    pltpu.sync_copy(s_loc, s_out.at[cid, sid])


def _sc_weights(el2, er2, m2, src, dst, zs):
    sds = jax.ShapeDtypeStruct
    return pl.kernel(
        _w_body,
        out_type=(sds((NC, NS, NBLK1, BLK), _f32), sds((NC, NS, N), _f32)),
        mesh=plsc.VectorSubcoreMesh(core_axis_name="c", subcore_axis_name="s"),
        compiler_params=pltpu.CompilerParams(needs_layout_passes=False),
        scratch_types=[
            pltpu.VMEM((N,), _f32),
            pltpu.VMEM((N,), _f32),
            pltpu.VMEM((L,), _f32),
            pltpu.VMEM((NBLK1, BLK), jnp.int32),
            pltpu.VMEM((NBLK1, BLK), jnp.int32),
            pltpu.VMEM((NBLK1, BLK), _f32),
            pltpu.VMEM((N,), _f32),
        ],
    )(el2, er2, m2, src, dst, zs)


def _a_body(tab_hbm, ex_hbm, src_hbm, dst_hbm, accin_hbm,
            acc_out,
            acc_sh, srcv, dstv, exv, rowsv, sem, ssem):
    cid = lax.axis_index("c")
    sid = lax.axis_index("s")
    wid = cid * NS + sid

    pltpu.sync_copy(src_hbm.at[wid], srcv)
    pltpu.sync_copy(dst_hbm.at[wid], dstv)
    pltpu.sync_copy(ex_hbm.at[wid], exv)

    @pl.when(sid == 0)
    def _restore():
        pltpu.sync_copy(accin_hbm.at[cid], acc_sh)

    plsc.subcore_barrier()

    pltpu.async_copy(tab_hbm.at[srcv.at[0]], rowsv.at[0], sem.at[0])

    def blk(j, carry):
        p = lax.rem(j, 2)
        pltpu.make_async_copy(tab_hbm.at[srcv.at[j]], rowsv.at[p],
                              sem.at[p]).wait()

        @pl.when(j + 1 < NBLK)
        def _prefetch():
            pltpu.async_copy(tab_hbm.at[srcv.at[j + 1]], rowsv.at[1 - p],
                             sem.at[1 - p])

        for k in range(BLK // L):
            for lq in range(L):
                e = k * L + lq
                bc = plsc.load_gather(
                    exv, [jnp.full((L,), j, jnp.int32),
                          jnp.full((L,), e, jnp.int32)])
                for col in range(F // L):
                    rowsv[p, e, pl.ds(col * L, L)] = (
                        rowsv[p, e, pl.ds(col * L, L)] * bc)
        pltpu.sync_copy(rowsv.at[p], acc_sh.at[dstv.at[j]], add=True)
        return carry

    lax.fori_loop(0, NBLK, blk, 0)

    plsc.subcore_barrier()

    @pl.when(sid == 0)
    def _flush():
        pltpu.sync_copy(acc_sh, acc_out.at[cid])


def _sc_accum(tab, exq, src, dst, accin):
    sds = jax.ShapeDtypeStruct
    return pl.kernel(
        _a_body,
        out_type=sds((NC, N, F), _f32),
        mesh=plsc.VectorSubcoreMesh(core_axis_name="c", subcore_axis_name="s"),
        compiler_params=pltpu.CompilerParams(needs_layout_passes=False),
        scratch_types=[
            pltpu.VMEM_SHARED((N, F), _f32),
            pltpu.VMEM((NBLK, BLK), jnp.int32),
            pltpu.VMEM((NBLK, BLK), jnp.int32),
            pltpu.VMEM((NBLK, BLK), _f32),
            pltpu.VMEM((2, BLK, F), _f32),
            pltpu.SemaphoreType.DMA((2,)),
            pltpu.SemaphoreType.DMA((2,)),
        ],
    )(tab, exq, src, dst, accin)


def _combine_body(acc1, s1, acc2, s2, bdd, btt, bx, hd, ht):
    def rel_term(acc_ref, s_ref, rel):
        sv = jnp.sum(s_ref[rel], axis=1).reshape(RB, 1)
        return jnp.where(sv > 0, acc_ref[rel] / sv, jnp.float32(0.0))

    hd[...] = (rel_term(acc1, s1, 0) + rel_term(acc2, s2, 0)
               + bdd[...] + bx[...])
    ht[...] = (rel_term(acc1, s1, 1) + rel_term(acc2, s2, 1)
               + btt[...] + bx[...])


def _combine(acc1, s1, acc2, s2, bdd, btt, bx):
    sds = jax.ShapeDtypeStruct
    acc_spec = pl.BlockSpec((NC, RB, F), lambda i: (0, i, 0))
    s_spec = pl.BlockSpec((NC, RB, NS), lambda i: (0, i, 0))
    b_spec = pl.BlockSpec((F,), lambda i: (0,))
    out_spec = pl.BlockSpec((RB, F), lambda i: (i, 0))
    return pl.pallas_call(
        _combine_body,
        grid=(N // RB,),
        in_specs=[acc_spec, s_spec, acc_spec, s_spec, b_spec, b_spec, b_spec],
        out_specs=(out_spec, out_spec),
        out_shape=(sds((N, F), _f32), sds((N, F), _f32)),
    )(acc1, s1, acc2, s2, bdd, btt, bx)


def _edges4(e_src, offset):
    e2 = jnp.pad(e_src.reshape(NS, NQ, EPT), ((0, 0), (0, 0), (0, EPTP - EPT)))
    return (e2 + jnp.int32(offset)).reshape(NS, NQ, NBLK, BLK)


def kernel(x_d, x_t, edge_dd, edge_tt, edge_dt, edge_td,
           W_dd, al_dd, ar_dd, b_dd,
           W_tt, al_tt, ar_tt, b_tt,
           W_x, al_x, ar_x, b_x):
    qa = jnp.stack([al_dd, ar_dd], axis=1)
    qc = jnp.stack([al_tt, ar_tt], axis=1)
    qx = jnp.stack([al_x, ar_x], axis=1)

    tab1, tab2, eea, eec, eeb, eed, mx = _prep(
        x_d, x_t, W_dd, W_tt, W_x, qa, qc, qx)
    tab1 = tab1.reshape(2 * N, F)
    tab2 = tab2.reshape(2 * N, F)

    el1 = jnp.stack([eea[:, 0], eec[:, 0]])
    er1 = jnp.stack([eea[:, 1], eec[:, 1]])
    m1 = jnp.stack([
        jnp.broadcast_to(mx[0, 0] + mx[1, 0], (L,)),
        jnp.broadcast_to(mx[2, 0] + mx[3, 0], (L,)),
    ])
    src1 = jnp.concatenate([_edges4(edge_dd[0], 0), _edges4(edge_tt[0], N)])
    dst1 = jnp.concatenate([_edges4(edge_dd[1], 0), _edges4(edge_tt[1], 0)])

    el2 = jnp.stack([eeb[:, 0], eed[:, 0]])
    er2 = jnp.stack([eed[:, 1], eeb[:, 1]])
    m2 = jnp.stack([
        jnp.broadcast_to(mx[4, 0] + mx[7, 0], (L,)),
        jnp.broadcast_to(mx[6, 0] + mx[5, 0], (L,)),
    ])
    src2 = jnp.concatenate([_edges4(edge_td[0], 0), _edges4(edge_dt[0], N)])
    dst2 = jnp.concatenate([_edges4(edge_td[1], 0), _edges4(edge_dt[1], 0)])

    zs = jnp.zeros((N,), _f32)
    zacc = jnp.zeros((NC, N, F), _f32)

    outs = []
    for tab, el, er, m, src, dst in ((tab1, el1, er1, m1, src1, dst1),
                                     (tab2, el2, er2, m2, src2, dst2)):
        if outs:
            pa, ps = outs[-1]
            tab, el, er, m, pa, ps = lax.optimization_barrier(
                (tab, el, er, m, pa, ps))
            outs[-1] = (pa, ps)
        exq, s_parts = _sc_weights(
            el, er, m,
            src.reshape(NC * NS, NBLK1, BLK),
            dst.reshape(NC * NS, NBLK1, BLK), zs)
        acc = zacc
        exq4 = exq.reshape(NC * NS, NQ, NBLK, BLK)
        src4 = src.reshape(NC * NS, NQ, NBLK, BLK)
        dst4 = dst.reshape(NC * NS, NQ, NBLK, BLK)
        for q in range(NQ):
            acc = _sc_accum(tab, exq4[:, q], src4[:, q], dst4[:, q], acc)
        outs.append((acc, s_parts))

    (acc1, s1), (acc2, s2) = outs
    s1 = s1.transpose(0, 2, 1)
    s2 = s2.transpose(0, 2, 1)
    return _combine(acc1, s1, acc2, s2, b_dd, b_tt, b_x)
